# baseline (speedup 1.0000x reference)
import functools
import jax
import jax.numpy as jnp
import numpy as np
from jax.experimental import pallas as pl
from jax.experimental.pallas import tpu as pltpu

_R_MAX = 5.0
_NB = 8
_PC = 5
_AVG = 16.0
_F = 16
_NE = 10
_NL = 2
_NG = 16

_C1 = float(np.sqrt(3.0))
_C2 = float(np.sqrt(15.0))
_C3 = float(np.sqrt(5.0) / 2.0)


def _silu(x):
    return x * jax.nn.sigmoid(x)


def _dsilu(x):
    s = jax.nn.sigmoid(x)
    return s * (1.0 + x * (1.0 - s))


def _radial(l):
    n = jnp.arange(1, _NB + 1, dtype=jnp.float32)[None, :]
    linv = 1.0 / (l + 1e-9)
    arg = n * (jnp.pi / _R_MAX) * l[:, None]
    s = jnp.sin(arg)
    c = jnp.cos(arg)
    pref = np.sqrt(2.0 / _R_MAX).astype(np.float32)
    bes = pref * s * linv[:, None]
    dbes = pref * (n * (jnp.pi / _R_MAX) * c * linv[:, None] - s * linv[:, None] ** 2)
    x = l / _R_MAX
    p = float(_PC)
    a = (p + 1.0) * (p + 2.0) / 2.0
    b = p * (p + 2.0)
    c2 = p * (p + 1.0) / 2.0
    f = 1.0 - a * x ** _PC + b * x ** (_PC + 1) - c2 * x ** (_PC + 2)
    df = (-a * _PC * x ** (_PC - 1) + b * (_PC + 1) * x ** _PC
          - c2 * (_PC + 2) * x ** (_PC + 1)) / _R_MAX
    inside = (x < 1.0)
    cut = jnp.where(inside, f, 0.0)
    dcut = jnp.where(inside, df, 0.0)
    ef = bes * cut[:, None]
    def_dl = dbes * cut[:, None] + bes * dcut[:, None]
    return ef, def_dl


def _sph(u):
    x, y, z = u[:, 0], u[:, 1], u[:, 2]
    return jnp.stack([jnp.ones_like(x), _C1 * x, _C1 * y, _C1 * z,
                      _C2 * x * y, _C2 * y * z, _C3 * (3.0 * z * z - 1.0),
                      _C2 * x * z, (_C2 / 2.0) * (x * x - y * y)], axis=-1)


def _sph_jt(u, g):
    x, y, z = u[:, 0], u[:, 1], u[:, 2]
    gx = _C1 * g[:, 1] + _C2 * (y * g[:, 4] + z * g[:, 7] + x * g[:, 8])
    gy = _C1 * g[:, 2] + _C2 * (x * g[:, 4] + z * g[:, 5] - y * g[:, 8])
    gz = _C1 * g[:, 3] + _C2 * (y * g[:, 5] + x * g[:, 7]) + 6.0 * _C3 * z * g[:, 6]
    return jnp.stack([gx, gy, gz], axis=-1)


_SEG_BLK = 4096


def _seg_kernel(x_ref, b_ref, o_ref):
    pid = pl.program_id(0)

    @pl.when(pid == 0)
    def _():
        o_ref[...] = jnp.zeros_like(o_ref)

    x = x_ref[...]
    b = b_ref[...]
    vals = jnp.stack([jnp.sum(jnp.where(b == g, x, 0.0)) for g in range(_NG)])
    o_ref[...] += vals[None, :]


def _seg16(x, batch_p, n_pad):
    xp = jnp.pad(x, (0, n_pad - x.shape[0]))
    x2 = xp.reshape(n_pad // 128, 128)
    b2 = batch_p.reshape(n_pad // 128, 128)
    rows = _SEG_BLK // 128
    grid = n_pad // _SEG_BLK
    out = pl.pallas_call(
        _seg_kernel,
        grid=(grid,),
        in_specs=[pl.BlockSpec((rows, 128), lambda i: (i, 0)),
                  pl.BlockSpec((rows, 128), lambda i: (i, 0))],
        out_specs=pl.BlockSpec((1, _NG), lambda i: (0, 0)),
        out_shape=jax.ShapeDtypeStruct((1, _NG), jnp.float32),
    )(x2, b2)
    return out[0]


def _forward_backward(positions, node_attrs, edge_index, shifts, batch,
                      atomic_energies, W_emb, W_up, W_r1, W_r2, W_down, W_sc,
                      Wp1, Wp2, Wp3, Wread0, Wm1, Wm2):
    N = positions.shape[0]
    s_idx, r_idx = edge_index[0], edge_index[1]
    order = jnp.argsort(r_idx)
    s_idx = s_idx[order]
    r_idx = r_idx[order]
    shifts = shifts[order]

    vec = positions[r_idx] - positions[s_idx] + shifts
    l = jnp.sqrt(jnp.sum(vec * vec, axis=-1) + 1e-9)
    u = vec / l[:, None]
    ef, def_dl = _radial(l)
    ea = _sph(u)

    node_e0 = node_attrs @ atomic_energies
    h0 = node_attrs @ W_emb

    hs = [h0]
    store = []
    for i in range(_NL):
        h = hs[-1]
        hu = h @ W_up[i]
        r1 = ef @ W_r1[i]
        w = _silu(r1) @ W_r2[i]
        hus = hu[s_idx]
        hj = hus * w
        m = ea[:, :, None] * hj[:, None, :]
        agg = jax.ops.segment_sum(m, r_idx, num_segments=N,
                                  indices_are_sorted=True) / _AVG
        aggd = jnp.einsum('nmf,fg->nmg', agg, W_down[i])
        sc = jnp.einsum('nf,na,afg->ng', h, node_attrs, W_sc[i])
        s1 = aggd[:, 0, :]
        s2 = jnp.sum(aggd * aggd, axis=1)
        hn = s1 @ Wp1[i] + s2 @ Wp2[i] + (s1 * s2) @ Wp3[i] + sc
        hs.append(hn)
        store.append((hu, r1, w, hus, hj, aggd, s1, s2))

    h1, h2 = hs[1], hs[2]
    a2 = h2 @ Wm1
    en1 = _silu(a2) @ Wm2
    en0 = h1 @ Wread0

    g_ea = jnp.zeros_like(ea)
    g_ef = jnp.zeros_like(ef)
    g_h = [jnp.zeros_like(h0), jnp.zeros_like(h0), jnp.zeros_like(h0)]
    g_h[2] = (_dsilu(a2) * Wm2[:, 0][None, :]) @ Wm1.T
    g_h[1] = jnp.broadcast_to(Wread0[:, 0][None, :], h1.shape)

    for i in range(_NL - 1, -1, -1):
        hu, r1, w, hus, hj, aggd, s1, s2 = store[i]
        G = g_h[i + 1]
        gp3 = G @ Wp3[i].T
        g_s1 = G @ Wp1[i].T + gp3 * s2
        g_s2 = G @ Wp2[i].T + gp3 * s1
        g_aggd = 2.0 * aggd * g_s2[:, None, :]
        g_aggd = g_aggd.at[:, 0, :].add(g_s1)
        g_agg = jnp.einsum('nmg,fg->nmf', g_aggd, W_down[i]) / _AVG
        g_h[i] = g_h[i] + jnp.einsum('ng,na,afg->nf', G, node_attrs, W_sc[i])
        g_m = g_agg[r_idx]
        g_hj = jnp.einsum('em,emf->ef', ea, g_m)
        g_ea = g_ea + jnp.einsum('emf,ef->em', g_m, hj)
        g_hus = g_hj * w
        g_w = g_hj * hus
        g_hu = jax.ops.segment_sum(g_hus, s_idx, num_segments=N)
        g_h[i] = g_h[i] + g_hu @ W_up[i].T
        g_ef = g_ef + (_dsilu(r1) * (g_w @ W_r2[i].T)) @ W_r1[i].T

    g_l = jnp.sum(g_ef * def_dl, axis=-1)
    g_u = _sph_jt(u, g_ea)
    g_vec = g_l[:, None] * u + (g_u - u * jnp.sum(u * g_u, axis=-1, keepdims=True)) / l[:, None]
    g_pos = jax.ops.segment_sum(g_vec, r_idx, num_segments=N,
                                indices_are_sorted=True) \
        - jax.ops.segment_sum(g_vec, s_idx, num_segments=N)
    forces = -g_pos

    return node_e0, en0[:, 0], en1[:, 0], forces


def kernel(positions, node_attrs, edge_index, shifts, batch, num_graphs,
           atomic_energies, W_emb, W_up, W_r1, W_r2, W_down, W_sc,
           Wp1, Wp2, Wp3, Wread0, Wm1, Wm2):
    N = positions.shape[0]
    node_e0, en0, en1, forces = _forward_backward(
        positions, node_attrs, edge_index, shifts, batch, atomic_energies,
        W_emb, W_up, W_r1, W_r2, W_down, W_sc, Wp1, Wp2, Wp3, Wread0, Wm1, Wm2)

    n_pad = ((N + _SEG_BLK - 1) // _SEG_BLK) * _SEG_BLK
    batch_p = jnp.pad(batch, (0, n_pad - N), constant_values=_NG)
    e0 = _seg16(node_e0, batch_p, n_pad)
    e1 = _seg16(en0, batch_p, n_pad)
    e2 = _seg16(en1, batch_p, n_pad)
    ng_zero = jnp.asarray(num_graphs, dtype=jnp.float32) * 0.0
    contributions = jnp.stack([e0 + ng_zero, e1, e2], axis=-1)
    total = jnp.sum(contributions, axis=-1)
    return total, contributions, forces


# speedup vs baseline: 1.1228x; 1.1228x over previous
import functools
import jax
import jax.numpy as jnp
import numpy as np
from jax.experimental import pallas as pl
from jax.experimental.pallas import tpu as pltpu

_R_MAX = 5.0
_NB = 8
_PC = 5
_AVG = 16.0
_F = 16
_NE = 10
_NL = 2
_NG = 16

_C1 = float(np.sqrt(3.0))
_C2 = float(np.sqrt(15.0))
_C3 = float(np.sqrt(5.0) / 2.0)


def _silu(x):
    return x * jax.nn.sigmoid(x)


def _dsilu(x):
    s = jax.nn.sigmoid(x)
    return s * (1.0 + x * (1.0 - s))


def _radial(l):
    n = jnp.arange(1, _NB + 1, dtype=jnp.float32)[None, :]
    linv = 1.0 / (l + 1e-9)
    arg = n * (jnp.pi / _R_MAX) * l[:, None]
    s = jnp.sin(arg)
    c = jnp.cos(arg)
    pref = np.sqrt(2.0 / _R_MAX).astype(np.float32)
    bes = pref * s * linv[:, None]
    dbes = pref * (n * (jnp.pi / _R_MAX) * c * linv[:, None] - s * linv[:, None] ** 2)
    x = l / _R_MAX
    p = float(_PC)
    a = (p + 1.0) * (p + 2.0) / 2.0
    b = p * (p + 2.0)
    c2 = p * (p + 1.0) / 2.0
    f = 1.0 - a * x ** _PC + b * x ** (_PC + 1) - c2 * x ** (_PC + 2)
    df = (-a * _PC * x ** (_PC - 1) + b * (_PC + 1) * x ** _PC
          - c2 * (_PC + 2) * x ** (_PC + 1)) / _R_MAX
    inside = (x < 1.0)
    cut = jnp.where(inside, f, 0.0)
    dcut = jnp.where(inside, df, 0.0)
    ef = bes * cut[:, None]
    def_dl = dbes * cut[:, None] + bes * dcut[:, None]
    return ef, def_dl


def _sph(u):
    x, y, z = u[:, 0], u[:, 1], u[:, 2]
    return jnp.stack([jnp.ones_like(x), _C1 * x, _C1 * y, _C1 * z,
                      _C2 * x * y, _C2 * y * z, _C3 * (3.0 * z * z - 1.0),
                      _C2 * x * z, (_C2 / 2.0) * (x * x - y * y)], axis=-1)


def _sph_jt(u, g):
    x, y, z = u[:, 0], u[:, 1], u[:, 2]
    gx = _C1 * g[:, 1] + _C2 * (y * g[:, 4] + z * g[:, 7] + x * g[:, 8])
    gy = _C1 * g[:, 2] + _C2 * (x * g[:, 4] + z * g[:, 5] - y * g[:, 8])
    gz = _C1 * g[:, 3] + _C2 * (y * g[:, 5] + x * g[:, 7]) + 6.0 * _C3 * z * g[:, 6]
    return jnp.stack([gx, gy, gz], axis=-1)


_SEG_BLK = 4096


def _seg_kernel(x_ref, b_ref, o_ref):
    pid = pl.program_id(0)

    @pl.when(pid == 0)
    def _():
        o_ref[...] = jnp.zeros_like(o_ref)

    x = x_ref[...]
    b = b_ref[...]
    vals = jnp.stack([jnp.sum(jnp.where(b == g, x, 0.0)) for g in range(_NG)])
    o_ref[...] += vals[None, :]


def _seg16(x, batch_p, n_pad):
    xp = jnp.pad(x, (0, n_pad - x.shape[0]))
    x2 = xp.reshape(n_pad // 128, 128)
    b2 = batch_p.reshape(n_pad // 128, 128)
    rows = _SEG_BLK // 128
    grid = n_pad // _SEG_BLK
    out = pl.pallas_call(
        _seg_kernel,
        grid=(grid,),
        in_specs=[pl.BlockSpec((rows, 128), lambda i: (i, 0)),
                  pl.BlockSpec((rows, 128), lambda i: (i, 0))],
        out_specs=pl.BlockSpec((1, _NG), lambda i: (0, 0)),
        out_shape=jax.ShapeDtypeStruct((1, _NG), jnp.float32),
    )(x2, b2)
    return out[0]


def _forward_backward(positions, node_attrs, edge_index, shifts, batch,
                      atomic_energies, W_emb, W_up, W_r1, W_r2, W_down, W_sc,
                      Wp1, Wp2, Wp3, Wread0, Wm1, Wm2):
    N = positions.shape[0]
    s_idx, r_idx = edge_index[0], edge_index[1]

    vec = positions[r_idx] - positions[s_idx] + shifts
    l = jnp.sqrt(jnp.sum(vec * vec, axis=-1) + 1e-9)
    u = vec / l[:, None]
    ef, def_dl = _radial(l)
    ea = _sph(u)

    node_e0 = node_attrs @ atomic_energies
    h0 = node_attrs @ W_emb

    hs = [h0]
    store = []
    for i in range(_NL):
        h = hs[-1]
        hu = h @ W_up[i]
        r1 = ef @ W_r1[i]
        w = _silu(r1) @ W_r2[i]
        hus = hu[s_idx]
        hj = hus * w
        m = ea[:, :, None] * hj[:, None, :]
        agg = jax.ops.segment_sum(m, r_idx, num_segments=N) / _AVG
        aggd = jnp.einsum('nmf,fg->nmg', agg, W_down[i])
        sc = jnp.einsum('nf,na,afg->ng', h, node_attrs, W_sc[i])
        s1 = aggd[:, 0, :]
        s2 = jnp.sum(aggd * aggd, axis=1)
        hn = s1 @ Wp1[i] + s2 @ Wp2[i] + (s1 * s2) @ Wp3[i] + sc
        hs.append(hn)
        store.append((hu, r1, w, hus, hj, aggd, s1, s2))

    h1, h2 = hs[1], hs[2]
    a2 = h2 @ Wm1
    en1 = _silu(a2) @ Wm2
    en0 = h1 @ Wread0

    g_ea = jnp.zeros_like(ea)
    g_ef = jnp.zeros_like(ef)
    g_h = [jnp.zeros_like(h0), jnp.zeros_like(h0), jnp.zeros_like(h0)]
    g_h[2] = (_dsilu(a2) * Wm2[:, 0][None, :]) @ Wm1.T
    g_h[1] = jnp.broadcast_to(Wread0[:, 0][None, :], h1.shape)

    for i in range(_NL - 1, -1, -1):
        hu, r1, w, hus, hj, aggd, s1, s2 = store[i]
        G = g_h[i + 1]
        gp3 = G @ Wp3[i].T
        g_s1 = G @ Wp1[i].T + gp3 * s2
        g_s2 = G @ Wp2[i].T + gp3 * s1
        g_aggd = 2.0 * aggd * g_s2[:, None, :]
        g_aggd = g_aggd.at[:, 0, :].add(g_s1)
        g_agg = jnp.einsum('nmg,fg->nmf', g_aggd, W_down[i]) / _AVG
        g_h[i] = g_h[i] + jnp.einsum('ng,na,afg->nf', G, node_attrs, W_sc[i])
        g_m = g_agg[r_idx]
        g_hj = jnp.einsum('em,emf->ef', ea, g_m)
        g_ea = g_ea + jnp.einsum('emf,ef->em', g_m, hj)
        g_hus = g_hj * w
        g_w = g_hj * hus
        g_hu = jax.ops.segment_sum(g_hus, s_idx, num_segments=N)
        g_h[i] = g_h[i] + g_hu @ W_up[i].T
        g_ef = g_ef + (_dsilu(r1) * (g_w @ W_r2[i].T)) @ W_r1[i].T

    g_l = jnp.sum(g_ef * def_dl, axis=-1)
    g_u = _sph_jt(u, g_ea)
    g_vec = g_l[:, None] * u + (g_u - u * jnp.sum(u * g_u, axis=-1, keepdims=True)) / l[:, None]
    g_pos = jax.ops.segment_sum(g_vec, r_idx, num_segments=N) \
        - jax.ops.segment_sum(g_vec, s_idx, num_segments=N)
    forces = -g_pos

    return node_e0, en0[:, 0], en1[:, 0], forces


def kernel(positions, node_attrs, edge_index, shifts, batch, num_graphs,
           atomic_energies, W_emb, W_up, W_r1, W_r2, W_down, W_sc,
           Wp1, Wp2, Wp3, Wread0, Wm1, Wm2):
    N = positions.shape[0]
    node_e0, en0, en1, forces = _forward_backward(
        positions, node_attrs, edge_index, shifts, batch, atomic_energies,
        W_emb, W_up, W_r1, W_r2, W_down, W_sc, Wp1, Wp2, Wp3, Wread0, Wm1, Wm2)

    n_pad = ((N + _SEG_BLK - 1) // _SEG_BLK) * _SEG_BLK
    batch_p = jnp.pad(batch, (0, n_pad - N), constant_values=_NG)
    e0 = _seg16(node_e0, batch_p, n_pad)
    e1 = _seg16(en0, batch_p, n_pad)
    e2 = _seg16(en1, batch_p, n_pad)
    ng_zero = jnp.asarray(num_graphs, dtype=jnp.float32) * 0.0
    contributions = jnp.stack([e0 + ng_zero, e1, e2], axis=-1)
    total = jnp.sum(contributions, axis=-1)
    return total, contributions, forces


# speedup vs baseline: 1.4326x; 1.2759x over previous
import functools
import jax
import jax.numpy as jnp
import numpy as np
from jax import lax
from jax.experimental import pallas as pl
from jax.experimental.pallas import tpu as pltpu
from jax.experimental.pallas import tpu_sc as plsc

_R_MAX = 5.0
_NB = 8
_PC = 5
_AVG = 16.0
_F = 16
_NE = 10
_NL = 2
_NG = 16

_C1 = float(np.sqrt(3.0))
_C2 = float(np.sqrt(15.0))
_C3 = float(np.sqrt(5.0) / 2.0)


def _silu(x):
    return x * jax.nn.sigmoid(x)


def _dsilu(x):
    s = jax.nn.sigmoid(x)
    return s * (1.0 + x * (1.0 - s))


def _radial(l):
    n = jnp.arange(1, _NB + 1, dtype=jnp.float32)[None, :]
    linv = 1.0 / (l + 1e-9)
    arg = n * (jnp.pi / _R_MAX) * l[:, None]
    s = jnp.sin(arg)
    c = jnp.cos(arg)
    pref = np.sqrt(2.0 / _R_MAX).astype(np.float32)
    bes = pref * s * linv[:, None]
    dbes = pref * (n * (jnp.pi / _R_MAX) * c * linv[:, None] - s * linv[:, None] ** 2)
    x = l / _R_MAX
    p = float(_PC)
    a = (p + 1.0) * (p + 2.0) / 2.0
    b = p * (p + 2.0)
    c2 = p * (p + 1.0) / 2.0
    f = 1.0 - a * x ** _PC + b * x ** (_PC + 1) - c2 * x ** (_PC + 2)
    df = (-a * _PC * x ** (_PC - 1) + b * (_PC + 1) * x ** _PC
          - c2 * (_PC + 2) * x ** (_PC + 1)) / _R_MAX
    inside = (x < 1.0)
    cut = jnp.where(inside, f, 0.0)
    dcut = jnp.where(inside, df, 0.0)
    ef = bes * cut[:, None]
    def_dl = dbes * cut[:, None] + bes * dcut[:, None]
    return ef, def_dl


def _sph(u):
    x, y, z = u[:, 0], u[:, 1], u[:, 2]
    return jnp.stack([jnp.ones_like(x), _C1 * x, _C1 * y, _C1 * z,
                      _C2 * x * y, _C2 * y * z, _C3 * (3.0 * z * z - 1.0),
                      _C2 * x * z, (_C2 / 2.0) * (x * x - y * y)], axis=-1)


def _sph_jt(u, g):
    x, y, z = u[:, 0], u[:, 1], u[:, 2]
    gx = _C1 * g[:, 1] + _C2 * (y * g[:, 4] + z * g[:, 7] + x * g[:, 8])
    gy = _C1 * g[:, 2] + _C2 * (x * g[:, 4] + z * g[:, 5] - y * g[:, 8])
    gz = _C1 * g[:, 3] + _C2 * (y * g[:, 5] + x * g[:, 7]) + 6.0 * _C3 * z * g[:, 6]
    return jnp.stack([gx, gy, gz], axis=-1)


_CHK = 128


@functools.lru_cache(maxsize=None)
def _make_gather(e_pad, n_rows, d):
    per = e_pad // 32
    chunks = per // _CHK
    mesh = plsc.VectorSubcoreMesh(core_axis_name="c", subcore_axis_name="s")

    @functools.partial(
        pl.kernel, mesh=mesh,
        out_type=jax.ShapeDtypeStruct((e_pad, d), jnp.float32),
        scratch_types=[
            pltpu.VMEM((_CHK,), jnp.int32),
            pltpu.VMEM((_CHK, d), jnp.float32),
            pltpu.SemaphoreType.DMA,
        ],
    )
    def k(table_hbm, idx_hbm, out_hbm, idx_v, rows_v, sem):
        c = lax.axis_index("c")
        s = lax.axis_index("s")
        wid = s * 2 + c

        def chunk(g, carry):
            base = wid * per + g * _CHK
            pltpu.sync_copy(idx_hbm.at[pl.ds(base, _CHK)], idx_v)
            pltpu.async_copy(table_hbm.at[idx_v], rows_v, sem).wait()
            pltpu.sync_copy(rows_v, out_hbm.at[pl.ds(base, _CHK)])
            return carry

        lax.fori_loop(0, chunks, chunk, 0)

    return k


def _sc_gather(table, idx):
    e = idx.shape[0]
    d = table.shape[1]
    d_pad = ((d + 127) // 128) * 128
    table = jnp.pad(table, ((0, 0), (0, d_pad - d)))
    e_pad = ((e + 32 * _CHK - 1) // (32 * _CHK)) * (32 * _CHK)
    idx_p = jnp.pad(idx, (0, e_pad - e))
    out = _make_gather(e_pad, table.shape[0], d_pad)(table, idx_p)
    return out[:e, :d]


_SEG_BLK = 4096


def _seg_kernel(x_ref, b_ref, o_ref):
    pid = pl.program_id(0)

    @pl.when(pid == 0)
    def _():
        o_ref[...] = jnp.zeros_like(o_ref)

    x = x_ref[...]
    b = b_ref[...]
    vals = jnp.stack([jnp.sum(jnp.where(b == g, x, 0.0)) for g in range(_NG)])
    o_ref[...] += vals[None, :]


def _seg16(x, batch_p, n_pad):
    xp = jnp.pad(x, (0, n_pad - x.shape[0]))
    x2 = xp.reshape(n_pad // 128, 128)
    b2 = batch_p.reshape(n_pad // 128, 128)
    rows = _SEG_BLK // 128
    grid = n_pad // _SEG_BLK
    out = pl.pallas_call(
        _seg_kernel,
        grid=(grid,),
        in_specs=[pl.BlockSpec((rows, 128), lambda i: (i, 0)),
                  pl.BlockSpec((rows, 128), lambda i: (i, 0))],
        out_specs=pl.BlockSpec((1, _NG), lambda i: (0, 0)),
        out_shape=jax.ShapeDtypeStruct((1, _NG), jnp.float32),
    )(x2, b2)
    return out[0]


def _forward_backward(positions, node_attrs, edge_index, shifts, batch,
                      atomic_energies, W_emb, W_up, W_r1, W_r2, W_down, W_sc,
                      Wp1, Wp2, Wp3, Wread0, Wm1, Wm2):
    N = positions.shape[0]
    s_idx, r_idx = edge_index[0], edge_index[1]

    vec = positions[r_idx] - positions[s_idx] + shifts
    l = jnp.sqrt(jnp.sum(vec * vec, axis=-1) + 1e-9)
    u = vec / l[:, None]
    ef, def_dl = _radial(l)
    ea = _sph(u)

    node_e0 = node_attrs @ atomic_energies
    h0 = node_attrs @ W_emb

    hs = [h0]
    store = []
    for i in range(_NL):
        h = hs[-1]
        hu = h @ W_up[i]
        r1 = ef @ W_r1[i]
        w = _silu(r1) @ W_r2[i]
        hus = hu[s_idx]
        hj = hus * w
        m = ea[:, :, None] * hj[:, None, :]
        agg = jax.ops.segment_sum(m, r_idx, num_segments=N) / _AVG
        aggd = jnp.einsum('nmf,fg->nmg', agg, W_down[i])
        sc = jnp.einsum('nf,na,afg->ng', h, node_attrs, W_sc[i])
        s1 = aggd[:, 0, :]
        s2 = jnp.sum(aggd * aggd, axis=1)
        hn = s1 @ Wp1[i] + s2 @ Wp2[i] + (s1 * s2) @ Wp3[i] + sc
        hs.append(hn)
        store.append((hu, r1, w, hus, hj, aggd, s1, s2))

    h1, h2 = hs[1], hs[2]
    a2 = h2 @ Wm1
    en1 = _silu(a2) @ Wm2
    en0 = h1 @ Wread0

    g_ea = jnp.zeros_like(ea)
    g_ef = jnp.zeros_like(ef)
    g_h = [jnp.zeros_like(h0), jnp.zeros_like(h0), jnp.zeros_like(h0)]
    g_h[2] = (_dsilu(a2) * Wm2[:, 0][None, :]) @ Wm1.T
    g_h[1] = jnp.broadcast_to(Wread0[:, 0][None, :], h1.shape)

    for i in range(_NL - 1, -1, -1):
        hu, r1, w, hus, hj, aggd, s1, s2 = store[i]
        G = g_h[i + 1]
        gp3 = G @ Wp3[i].T
        g_s1 = G @ Wp1[i].T + gp3 * s2
        g_s2 = G @ Wp2[i].T + gp3 * s1
        g_aggd = 2.0 * aggd * g_s2[:, None, :]
        g_aggd = g_aggd.at[:, 0, :].add(g_s1)
        g_agg = jnp.einsum('nmg,fg->nmf', g_aggd, W_down[i]) / _AVG
        g_h[i] = g_h[i] + jnp.einsum('ng,na,afg->nf', G, node_attrs, W_sc[i])
        g_m = _sc_gather(g_agg.reshape(N, 144), r_idx).reshape(-1, 9, 16)
        g_hj = jnp.einsum('em,emf->ef', ea, g_m)
        g_ea = g_ea + jnp.einsum('emf,ef->em', g_m, hj)
        g_hus = g_hj * w
        g_w = g_hj * hus
        g_hu = jax.ops.segment_sum(g_hus, s_idx, num_segments=N)
        g_h[i] = g_h[i] + g_hu @ W_up[i].T
        g_ef = g_ef + (_dsilu(r1) * (g_w @ W_r2[i].T)) @ W_r1[i].T

    g_l = jnp.sum(g_ef * def_dl, axis=-1)
    g_u = _sph_jt(u, g_ea)
    g_vec = g_l[:, None] * u + (g_u - u * jnp.sum(u * g_u, axis=-1, keepdims=True)) / l[:, None]
    g_pos = jax.ops.segment_sum(g_vec, r_idx, num_segments=N) \
        - jax.ops.segment_sum(g_vec, s_idx, num_segments=N)
    forces = -g_pos

    return node_e0, en0[:, 0], en1[:, 0], forces


def kernel(positions, node_attrs, edge_index, shifts, batch, num_graphs,
           atomic_energies, W_emb, W_up, W_r1, W_r2, W_down, W_sc,
           Wp1, Wp2, Wp3, Wread0, Wm1, Wm2):
    N = positions.shape[0]
    node_e0, en0, en1, forces = _forward_backward(
        positions, node_attrs, edge_index, shifts, batch, atomic_energies,
        W_emb, W_up, W_r1, W_r2, W_down, W_sc, Wp1, Wp2, Wp3, Wread0, Wm1, Wm2)

    n_pad = ((N + _SEG_BLK - 1) // _SEG_BLK) * _SEG_BLK
    batch_p = jnp.pad(batch, (0, n_pad - N), constant_values=_NG)
    e0 = _seg16(node_e0, batch_p, n_pad)
    e1 = _seg16(en0, batch_p, n_pad)
    e2 = _seg16(en1, batch_p, n_pad)
    ng_zero = jnp.asarray(num_graphs, dtype=jnp.float32) * 0.0
    contributions = jnp.stack([e0 + ng_zero, e1, e2], axis=-1)
    total = jnp.sum(contributions, axis=-1)
    return total, contributions, forces


# speedup vs baseline: 1.4376x; 1.0035x over previous
import functools
import jax
import jax.numpy as jnp
import numpy as np
from jax import lax
from jax.experimental import pallas as pl
from jax.experimental.pallas import tpu as pltpu
from jax.experimental.pallas import tpu_sc as plsc

_R_MAX = 5.0
_NB = 8
_PC = 5
_AVG = 16.0
_F = 16
_NE = 10
_NL = 2
_NG = 16

_C1 = float(np.sqrt(3.0))
_C2 = float(np.sqrt(15.0))
_C3 = float(np.sqrt(5.0) / 2.0)


def _silu(x):
    return x * jax.nn.sigmoid(x)


def _dsilu(x):
    s = jax.nn.sigmoid(x)
    return s * (1.0 + x * (1.0 - s))


def _radial(l):
    n = jnp.arange(1, _NB + 1, dtype=jnp.float32)[None, :]
    linv = 1.0 / (l + 1e-9)
    arg = n * (jnp.pi / _R_MAX) * l[:, None]
    s = jnp.sin(arg)
    c = jnp.cos(arg)
    pref = np.sqrt(2.0 / _R_MAX).astype(np.float32)
    bes = pref * s * linv[:, None]
    dbes = pref * (n * (jnp.pi / _R_MAX) * c * linv[:, None] - s * linv[:, None] ** 2)
    x = l / _R_MAX
    p = float(_PC)
    a = (p + 1.0) * (p + 2.0) / 2.0
    b = p * (p + 2.0)
    c2 = p * (p + 1.0) / 2.0
    f = 1.0 - a * x ** _PC + b * x ** (_PC + 1) - c2 * x ** (_PC + 2)
    df = (-a * _PC * x ** (_PC - 1) + b * (_PC + 1) * x ** _PC
          - c2 * (_PC + 2) * x ** (_PC + 1)) / _R_MAX
    inside = (x < 1.0)
    cut = jnp.where(inside, f, 0.0)
    dcut = jnp.where(inside, df, 0.0)
    ef = bes * cut[:, None]
    def_dl = dbes * cut[:, None] + bes * dcut[:, None]
    return ef, def_dl


def _sph(u):
    x, y, z = u[:, 0], u[:, 1], u[:, 2]
    return jnp.stack([jnp.ones_like(x), _C1 * x, _C1 * y, _C1 * z,
                      _C2 * x * y, _C2 * y * z, _C3 * (3.0 * z * z - 1.0),
                      _C2 * x * z, (_C2 / 2.0) * (x * x - y * y)], axis=-1)


def _sph_jt(u, g):
    x, y, z = u[:, 0], u[:, 1], u[:, 2]
    gx = _C1 * g[:, 1] + _C2 * (y * g[:, 4] + z * g[:, 7] + x * g[:, 8])
    gy = _C1 * g[:, 2] + _C2 * (x * g[:, 4] + z * g[:, 5] - y * g[:, 8])
    gz = _C1 * g[:, 3] + _C2 * (y * g[:, 5] + x * g[:, 7]) + 6.0 * _C3 * z * g[:, 6]
    return jnp.stack([gx, gy, gz], axis=-1)


_CHK = 128


@functools.lru_cache(maxsize=None)
def _make_gather(e_pad, n_rows, d):
    per = e_pad // 32
    chunks = per // _CHK
    mesh = plsc.VectorSubcoreMesh(core_axis_name="c", subcore_axis_name="s")

    @functools.partial(
        pl.kernel, mesh=mesh,
        out_type=jax.ShapeDtypeStruct((e_pad, d), jnp.float32),
        scratch_types=[
            pltpu.VMEM((_CHK,), jnp.int32),
            pltpu.VMEM((_CHK, d), jnp.float32),
            pltpu.SemaphoreType.DMA,
        ],
    )
    def k(table_hbm, idx_hbm, out_hbm, idx_v, rows_v, sem):
        c = lax.axis_index("c")
        s = lax.axis_index("s")
        wid = s * 2 + c

        def chunk(g, carry):
            base = wid * per + g * _CHK
            pltpu.sync_copy(idx_hbm.at[pl.ds(base, _CHK)], idx_v)
            pltpu.async_copy(table_hbm.at[idx_v], rows_v, sem).wait()
            pltpu.sync_copy(rows_v, out_hbm.at[pl.ds(base, _CHK)])
            return carry

        lax.fori_loop(0, chunks, chunk, 0)

    return k


def _sc_gather(table, idx):
    e = idx.shape[0]
    d = table.shape[1]
    d_pad = ((d + 127) // 128) * 128
    table = jnp.pad(table, ((0, 0), (0, d_pad - d)))
    e_pad = ((e + 32 * _CHK - 1) // (32 * _CHK)) * (32 * _CHK)
    idx_p = jnp.pad(idx, (0, e_pad - e))
    out = _make_gather(e_pad, table.shape[0], d_pad)(table, idx_p)
    return out[:e, :d]


_SEG_BLK = 4096


def _seg_kernel(x_ref, b_ref, o_ref):
    pid = pl.program_id(0)

    @pl.when(pid == 0)
    def _():
        o_ref[...] = jnp.zeros_like(o_ref)

    x = x_ref[...]
    b = b_ref[...]
    vals = jnp.stack([jnp.sum(jnp.where(b == g, x, 0.0)) for g in range(_NG)])
    o_ref[...] += vals[None, :]


def _seg16(x, batch_p, n_pad):
    xp = jnp.pad(x, (0, n_pad - x.shape[0]))
    x2 = xp.reshape(n_pad // 128, 128)
    b2 = batch_p.reshape(n_pad // 128, 128)
    rows = _SEG_BLK // 128
    grid = n_pad // _SEG_BLK
    out = pl.pallas_call(
        _seg_kernel,
        grid=(grid,),
        in_specs=[pl.BlockSpec((rows, 128), lambda i: (i, 0)),
                  pl.BlockSpec((rows, 128), lambda i: (i, 0))],
        out_specs=pl.BlockSpec((1, _NG), lambda i: (0, 0)),
        out_shape=jax.ShapeDtypeStruct((1, _NG), jnp.float32),
    )(x2, b2)
    return out[0]


def _forward_backward(positions, node_attrs, edge_index, shifts, batch,
                      atomic_energies, W_emb, W_up, W_r1, W_r2, W_down, W_sc,
                      Wp1, Wp2, Wp3, Wread0, Wm1, Wm2):
    N = positions.shape[0]
    s_idx, r_idx = edge_index[0], edge_index[1]

    vec = positions[r_idx] - positions[s_idx] + shifts
    l = jnp.sqrt(jnp.sum(vec * vec, axis=-1) + 1e-9)
    u = vec / l[:, None]
    ef, def_dl = _radial(l)
    ea = _sph(u)

    node_e0 = node_attrs @ atomic_energies
    h0 = node_attrs @ W_emb

    hs = [h0]
    store = []
    for i in range(_NL):
        h = hs[-1]
        hu = h @ W_up[i]
        r1 = ef @ W_r1[i]
        w = _silu(r1) @ W_r2[i]
        hus = _sc_gather(hu, s_idx)
        hj = hus * w
        m = ea[:, :, None] * hj[:, None, :]
        agg = jax.ops.segment_sum(m, r_idx, num_segments=N) / _AVG
        aggd = jnp.einsum('nmf,fg->nmg', agg, W_down[i])
        sc = jnp.einsum('nf,na,afg->ng', h, node_attrs, W_sc[i])
        s1 = aggd[:, 0, :]
        s2 = jnp.sum(aggd * aggd, axis=1)
        hn = s1 @ Wp1[i] + s2 @ Wp2[i] + (s1 * s2) @ Wp3[i] + sc
        hs.append(hn)
        store.append((hu, r1, w, hus, hj, aggd, s1, s2))

    h1, h2 = hs[1], hs[2]
    a2 = h2 @ Wm1
    en1 = _silu(a2) @ Wm2
    en0 = h1 @ Wread0

    g_ea = jnp.zeros_like(ea)
    g_ef = jnp.zeros_like(ef)
    g_h = [jnp.zeros_like(h0), jnp.zeros_like(h0), jnp.zeros_like(h0)]
    g_h[2] = (_dsilu(a2) * Wm2[:, 0][None, :]) @ Wm1.T
    g_h[1] = jnp.broadcast_to(Wread0[:, 0][None, :], h1.shape)

    for i in range(_NL - 1, -1, -1):
        hu, r1, w, hus, hj, aggd, s1, s2 = store[i]
        G = g_h[i + 1]
        gp3 = G @ Wp3[i].T
        g_s1 = G @ Wp1[i].T + gp3 * s2
        g_s2 = G @ Wp2[i].T + gp3 * s1
        g_aggd = 2.0 * aggd * g_s2[:, None, :]
        g_aggd = g_aggd.at[:, 0, :].add(g_s1)
        g_agg = jnp.einsum('nmg,fg->nmf', g_aggd, W_down[i]) / _AVG
        g_h[i] = g_h[i] + jnp.einsum('ng,na,afg->nf', G, node_attrs, W_sc[i])
        g_m = _sc_gather(g_agg.reshape(N, 144), r_idx).reshape(-1, 9, 16)
        g_hj = jnp.einsum('em,emf->ef', ea, g_m)
        g_ea = g_ea + jnp.einsum('emf,ef->em', g_m, hj)
        g_hus = g_hj * w
        g_w = g_hj * hus
        g_hu = jax.ops.segment_sum(g_hus, s_idx, num_segments=N)
        g_h[i] = g_h[i] + g_hu @ W_up[i].T
        g_ef = g_ef + (_dsilu(r1) * (g_w @ W_r2[i].T)) @ W_r1[i].T

    g_l = jnp.sum(g_ef * def_dl, axis=-1)
    g_u = _sph_jt(u, g_ea)
    g_vec = g_l[:, None] * u + (g_u - u * jnp.sum(u * g_u, axis=-1, keepdims=True)) / l[:, None]
    g_pos = jax.ops.segment_sum(g_vec, r_idx, num_segments=N) \
        - jax.ops.segment_sum(g_vec, s_idx, num_segments=N)
    forces = -g_pos

    return node_e0, en0[:, 0], en1[:, 0], forces


def kernel(positions, node_attrs, edge_index, shifts, batch, num_graphs,
           atomic_energies, W_emb, W_up, W_r1, W_r2, W_down, W_sc,
           Wp1, Wp2, Wp3, Wread0, Wm1, Wm2):
    N = positions.shape[0]
    node_e0, en0, en1, forces = _forward_backward(
        positions, node_attrs, edge_index, shifts, batch, atomic_energies,
        W_emb, W_up, W_r1, W_r2, W_down, W_sc, Wp1, Wp2, Wp3, Wread0, Wm1, Wm2)

    n_pad = ((N + _SEG_BLK - 1) // _SEG_BLK) * _SEG_BLK
    batch_p = jnp.pad(batch, (0, n_pad - N), constant_values=_NG)
    e0 = _seg16(node_e0, batch_p, n_pad)
    e1 = _seg16(en0, batch_p, n_pad)
    e2 = _seg16(en1, batch_p, n_pad)
    ng_zero = jnp.asarray(num_graphs, dtype=jnp.float32) * 0.0
    contributions = jnp.stack([e0 + ng_zero, e1, e2], axis=-1)
    total = jnp.sum(contributions, axis=-1)
    return total, contributions, forces
